# Initial kernel scaffold; baseline (speedup 1.0000x reference)
import jax
import jax.numpy as jnp
from jax.experimental import pallas as pl


def kernel(states, W1, b1, W2, b2):
    raise NotImplementedError("write your pallas kernel here")


# speedup vs baseline: 1.1689x; 1.1689x over previous
import functools

import jax
import jax.numpy as jnp
from jax.experimental import pallas as pl
from jax.experimental.pallas import tpu as pltpu

_BATCH = 16384
_STATE_DIM = 256
_HIDDEN = 1024
_NUM_ACTIONS = 257
_PAD_ACTIONS = 384
_ROWS = 512


def _fused_body(s_ref, w1_ref, b1_ref, w2_ref, b2_ref, o_ref):
    s = s_ref[...]
    h = jnp.dot(s.astype(jnp.bfloat16), w1_ref[...],
                preferred_element_type=jnp.float32)
    h = jnp.maximum(h + b1_ref[...], 0.0)
    logits = jnp.dot(h.astype(jnp.bfloat16), w2_ref[...],
                     preferred_element_type=jnp.float32)
    logits = logits + b2_ref[...]
    mx = jnp.max(logits, axis=1, keepdims=True)
    e = jnp.exp(logits - mx)
    cont = (s < 2.0).astype(jnp.float32)
    col = jax.lax.broadcasted_iota(jnp.int32, (s.shape[0], 128), 1)
    tail = (col == 0).astype(jnp.float32)
    mask = jnp.concatenate([cont, tail], axis=1)
    me = e * mask
    out = me / jnp.sum(me, axis=1, keepdims=True)
    o_ref[...] = out[:, :_NUM_ACTIONS]


@functools.partial(jax.jit, static_argnames=())
def kernel(states, W1, b1, W2, b2):
    w1 = W1.astype(jnp.bfloat16)
    w2 = jnp.pad(W2, ((0, 0), (0, _PAD_ACTIONS - _NUM_ACTIONS))).astype(
        jnp.bfloat16)
    b1r = b1.reshape(1, _HIDDEN)
    b2r = jnp.pad(b2, (0, _PAD_ACTIONS - _NUM_ACTIONS),
                  constant_values=-1e9).reshape(1, _PAD_ACTIONS)
    grid = (_BATCH // _ROWS,)
    return pl.pallas_call(
        _fused_body,
        grid=grid,
        in_specs=[
            pl.BlockSpec((_ROWS, _STATE_DIM), lambda i: (i, 0)),
            pl.BlockSpec((_STATE_DIM, _HIDDEN), lambda i: (0, 0)),
            pl.BlockSpec((1, _HIDDEN), lambda i: (0, 0)),
            pl.BlockSpec((_HIDDEN, _PAD_ACTIONS), lambda i: (0, 0)),
            pl.BlockSpec((1, _PAD_ACTIONS), lambda i: (0, 0)),
        ],
        out_specs=pl.BlockSpec((_ROWS, _NUM_ACTIONS), lambda i: (i, 0)),
        out_shape=jax.ShapeDtypeStruct((_BATCH, _NUM_ACTIONS), jnp.float32),
        compiler_params=pltpu.CompilerParams(
            dimension_semantics=("parallel",),
        ),
    )(states, w1, b1r, w2, b2r)
